# Initial kernel scaffold; baseline (speedup 1.0000x reference)
import jax
import jax.numpy as jnp
from jax.experimental import pallas as pl


def kernel(x, edge_index, t, time_W0, time_b0, time_W1, time_b1, l0_Wl, l0_bl, l0_Wr, l0_br, l0_att, l0_bias, l0_res_W, l0_res_b, l1_Wl, l1_Wr, l1_att, l1_res_W, l1_res_b, dec_W, dec_b, cls_W, cls_b):
    raise NotImplementedError("write your pallas kernel here")


# speedup vs baseline: 29.7009x; 29.7009x over previous
import dataclasses
import functools

import jax
import jax.numpy as jnp
from jax import lax
from jax.experimental import pallas as pl
from jax.experimental.pallas import tpu as pltpu
from jax.experimental.pallas import tpu_sc as plsc

N = 100000
E = 1600000
NF = 16
TDIM = 16
H = 2
C = 16
HID = H * C

NSUB = 16
EPT = E // NSUB
CH = 80
NCH = EPT // CH
ROWS_PER_TILE = 6248
TAIL_ROWS = N - NSUB * ROWS_PER_TILE
TAIL_OFF = NSUB * ROWS_PER_TILE


def _mesh():
    return plsc.VectorSubcoreMesh(core_axis_name="c", subcore_axis_name="s")


def _sc_params():
    cp = pltpu.CompilerParams()
    if "needs_layout_passes" in pltpu.CompilerParams.__dataclass_fields__:
        cp = dataclasses.replace(cp, needs_layout_passes=False)
    if "use_tc_tiling_on_sc" in pltpu.CompilerParams.__dataclass_fields__:
        cp = dataclasses.replace(cp, use_tc_tiling_on_sc=False)
    return cp


def _sc_phase_a(xlst, xrst, src, dst, attf, zeros):
    @functools.partial(
        pl.kernel,
        mesh=_mesh(),
        compiler_params=_sc_params(),
        out_type=[
            jax.ShapeDtypeStruct((2 * E,), jnp.float32),
            jax.ShapeDtypeStruct((2 * N, C), jnp.float32),
        ],
        scratch_types=[
            pltpu.VMEM((CH,), jnp.int32),
            pltpu.VMEM((CH,), jnp.int32),
            pltpu.VMEM((CH,), jnp.int32),
            pltpu.VMEM((CH, C), jnp.float32),
            pltpu.VMEM((CH, C), jnp.float32),
            pltpu.VMEM((CH, C), jnp.float32),
            pltpu.VMEM((CH,), jnp.float32),
            pltpu.VMEM((C,), jnp.float32),
            pltpu.VMEM_SHARED((N, C), jnp.float32),
            pltpu.SemaphoreType.DMA,
            pltpu.SemaphoreType.DMA,
        ],
    )
    def kern(xl_hbm, xr_hbm, src_hbm, dst_hbm, att_hbm, z_hbm, ex_hbm, den_hbm,
             srcb, dstb, dstbo, xlb, xrb, denb, exb, attv, spden, sem1, sem2):
        cid = lax.axis_index("c")
        sid = lax.axis_index("s")
        hoff = cid * N

        pltpu.sync_copy(z_hbm, spden.at[pl.ds(sid * ROWS_PER_TILE, ROWS_PER_TILE)])

        @pl.when(sid == NSUB - 1)
        def _():
            pltpu.sync_copy(z_hbm.at[pl.ds(0, TAIL_ROWS)],
                            spden.at[pl.ds(TAIL_OFF, TAIL_ROWS)])

        pltpu.sync_copy(att_hbm.at[pl.ds(cid * C, C)], attv)
        plsc.subcore_barrier()

        iota16 = lax.iota(jnp.int32, 16)
        att_sp = [plsc.load_gather(attv, [jnp.full((16,), c, jnp.int32)])
                  for c in range(C)]

        @pl.loop(0, NCH)
        def _(ci):
            ebase = sid * EPT + ci * CH
            pltpu.sync_copy(src_hbm.at[pl.ds(ebase, CH)], srcb)
            pltpu.sync_copy(dst_hbm.at[pl.ds(ebase, CH)], dstb)
            for g in range(CH // 16):
                sl = pl.ds(g * 16, 16)
                srcb[sl] = srcb[sl] + hoff
                dstbo[sl] = dstb[sl] + hoff
            cp1 = pltpu.async_copy(xl_hbm.at[srcb], xlb, sem1)
            cp2 = pltpu.async_copy(xr_hbm.at[dstbo], xrb, sem2)
            cp1.wait()
            cp2.wait()
            for g in range(CH // 16):
                edge16 = iota16 + g * 16
                acc = jnp.zeros((16,), jnp.float32)
                for c in range(C):
                    csp = jnp.full((16,), c, jnp.int32)
                    xlv = plsc.load_gather(xlb, [edge16, csp])
                    xrv = plsc.load_gather(xrb, [edge16, csp])
                    sv = xlv + xrv
                    lv = jnp.maximum(sv, 0.2 * sv)
                    acc = acc + lv * att_sp[c]
                ex16 = jnp.exp(acc)
                exb[pl.ds(g * 16, 16)] = ex16
                for e in range(16):
                    esp = jnp.full((16,), g * 16 + e, jnp.int32)
                    denb[g * 16 + e, :] = plsc.load_gather(exb, [esp])
            pltpu.sync_copy(denb, spden.at[dstb], add=True)
            pltpu.sync_copy(exb, ex_hbm.at[pl.ds(cid * E + ebase, CH)])

        plsc.subcore_barrier()
        r0 = sid * ROWS_PER_TILE
        pltpu.sync_copy(spden.at[pl.ds(r0, ROWS_PER_TILE)],
                        den_hbm.at[pl.ds(cid * N + r0, ROWS_PER_TILE)])

        @pl.when(sid == NSUB - 1)
        def _():
            pltpu.sync_copy(spden.at[pl.ds(TAIL_OFF, TAIL_ROWS)],
                            den_hbm.at[pl.ds(cid * N + TAIL_OFF, TAIL_ROWS)])

    return kern(xlst, xrst, src, dst, attf, zeros)


def _sc_phase_b(xlst, src, dst, ex, zeros):
    @functools.partial(
        pl.kernel,
        mesh=_mesh(),
        compiler_params=_sc_params(),
        out_type=jax.ShapeDtypeStruct((2 * N, C), jnp.float32),
        scratch_types=[
            pltpu.VMEM((CH,), jnp.int32),
            pltpu.VMEM((CH,), jnp.int32),
            pltpu.VMEM((CH, C), jnp.float32),
            pltpu.VMEM((CH, C), jnp.float32),
            pltpu.VMEM((CH,), jnp.float32),
            pltpu.VMEM_SHARED((N, C), jnp.float32),
            pltpu.SemaphoreType.DMA,
        ],
    )
    def kern(xl_hbm, src_hbm, dst_hbm, ex_hbm, z_hbm, num_hbm,
             srcb, dstb, xlb, numb, exb, spnum, sem1):
        cid = lax.axis_index("c")
        sid = lax.axis_index("s")
        hoff = cid * N

        pltpu.sync_copy(z_hbm, spnum.at[pl.ds(sid * ROWS_PER_TILE, ROWS_PER_TILE)])

        @pl.when(sid == NSUB - 1)
        def _():
            pltpu.sync_copy(z_hbm.at[pl.ds(0, TAIL_ROWS)],
                            spnum.at[pl.ds(TAIL_OFF, TAIL_ROWS)])

        plsc.subcore_barrier()

        iota16 = lax.iota(jnp.int32, 16)

        @pl.loop(0, NCH)
        def _(ci):
            ebase = sid * EPT + ci * CH
            pltpu.sync_copy(src_hbm.at[pl.ds(ebase, CH)], srcb)
            pltpu.sync_copy(dst_hbm.at[pl.ds(ebase, CH)], dstb)
            pltpu.sync_copy(ex_hbm.at[pl.ds(cid * E + ebase, CH)], exb)
            for g in range(CH // 16):
                sl = pl.ds(g * 16, 16)
                srcb[sl] = srcb[sl] + hoff
            pltpu.async_copy(xl_hbm.at[srcb], xlb, sem1).wait()
            for e in range(CH):
                esp = jnp.full((16,), e, jnp.int32)
                vsp = plsc.load_gather(exb, [esp])
                row = plsc.load_gather(xlb, [esp, iota16])
                numb[e, :] = row * vsp
            pltpu.sync_copy(numb, spnum.at[dstb], add=True)

        plsc.subcore_barrier()
        r0 = sid * ROWS_PER_TILE
        pltpu.sync_copy(spnum.at[pl.ds(r0, ROWS_PER_TILE)],
                        num_hbm.at[pl.ds(cid * N + r0, ROWS_PER_TILE)])

        @pl.when(sid == NSUB - 1)
        def _():
            pltpu.sync_copy(spnum.at[pl.ds(TAIL_OFF, TAIL_ROWS)],
                            num_hbm.at[pl.ds(cid * N + TAIL_OFF, TAIL_ROWS)])

    return kern(xlst, src, dst, ex, zeros)


R = 2000
NB = N // R

_full = lambda shape: pl.BlockSpec(shape, lambda i: tuple(0 for _ in shape))
_rows32 = pl.BlockSpec((R, HID), lambda i: (i, 0))
_rows16 = pl.BlockSpec((R, NF), lambda i: (i, 0))
_st16 = pl.BlockSpec((2, R, C), lambda i: (0, i, 0))


def _prep_body(tf, w0, b0, w1, b1, wlt, bl, wrt, br, wrest, bres,
               cl_ref, cr_ref, cres_ref):
    ts = tf[0, 0]
    i8 = lax.broadcasted_iota(jnp.int32, (1, TDIM // 2), 1).astype(jnp.float32)
    freq = jnp.exp(i8 * (-jnp.log(10000.0) / (TDIM // 2 - 1)))
    emb = ts * freq
    emb = jnp.concatenate([jnp.sin(emb), jnp.cos(emb)], axis=1)
    temb = jax.nn.silu(jnp.dot(emb, w0[...], preferred_element_type=jnp.float32)
                       + b0[...])
    temb = jax.nn.silu(jnp.dot(temb, w1[...], preferred_element_type=jnp.float32)
                       + b1[...])
    cl_ref[...] = bl[...] + jnp.dot(temb, wlt[...], preferred_element_type=jnp.float32)
    cr_ref[...] = br[...] + jnp.dot(temb, wrt[...], preferred_element_type=jnp.float32)
    cres_ref[...] = bres[...] + jnp.dot(temb, wrest[...],
                                        preferred_element_type=jnp.float32)


def _self_terms(xl, xr, attf):
    s = xl + xr
    lv = jnp.maximum(s, 0.2 * s) * attf
    e0 = jnp.exp(jnp.sum(lv[:, :C], axis=1, keepdims=True))
    e1 = jnp.exp(jnp.sum(lv[:, C:], axis=1, keepdims=True))
    return e0, e1


def _node0_body(x, wlx, wrx, wresx, cl, cr, cres, attf,
                xlst, xrst, res0, snum, sden):
    xv = x[...]
    xl = jnp.dot(xv, wlx[...], preferred_element_type=jnp.float32) + cl[...]
    xr = jnp.dot(xv, wrx[...], preferred_element_type=jnp.float32) + cr[...]
    res0[...] = jax.nn.relu(
        jnp.dot(xv, wresx[...], preferred_element_type=jnp.float32) + cres[...])
    e0, e1 = _self_terms(xl, xr, attf[...])
    xlst[0] = xl[:, :C]
    xlst[1] = xl[:, C:]
    xrst[0] = xr[:, :C]
    xrst[1] = xr[:, C:]
    snum[0] = xl[:, :C] * e0
    snum[1] = xl[:, C:] * e1
    sden[0] = jnp.broadcast_to(e0, (R, C))
    sden[1] = jnp.broadcast_to(e1, (R, C))


def _node1_body(num, den, snum, sden, res0, bias0, wl, wr, wres, bres, attf,
                xlst, xrst, res1, snum1, sden1):
    o0 = (num[0] + snum[0]) / (den[0] + sden[0])
    o1 = (num[1] + snum[1]) / (den[1] + sden[1])
    hc = jax.nn.relu(jnp.concatenate([o0, o1], axis=1) + bias0[...])
    h = hc + res0[...]
    xl = jnp.dot(h, wl[...], preferred_element_type=jnp.float32)
    xr = jnp.dot(h, wr[...], preferred_element_type=jnp.float32)
    res1[...] = jax.nn.relu(
        jnp.dot(h, wres[...], preferred_element_type=jnp.float32) + bres[...])
    e0, e1 = _self_terms(xl, xr, attf[...])
    xlst[0] = xl[:, :C]
    xlst[1] = xl[:, C:]
    xrst[0] = xr[:, :C]
    xrst[1] = xr[:, C:]
    snum1[0] = xl[:, :C] * e0
    snum1[1] = xl[:, C:] * e1
    sden1[0] = jnp.broadcast_to(e0, (R, C))
    sden1[1] = jnp.broadcast_to(e1, (R, C))


def _decode_body(num, den, snum, sden, res1, decw, decb, clsw, clsb, out):
    o0 = (num[0] + snum[0]) / (den[0] + sden[0])
    o1 = (num[1] + snum[1]) / (den[1] + sden[1])
    h = jax.nn.relu(jnp.concatenate([o0, o1], axis=1)) + res1[...]
    d = jnp.dot(h, decw[...], preferred_element_type=jnp.float32) + decb[...]
    out[...] = jnp.dot(d, clsw[...], preferred_element_type=jnp.float32) + clsb[...]


def kernel(x, edge_index, t, time_W0, time_b0, time_W1, time_b1,
           l0_Wl, l0_bl, l0_Wr, l0_br, l0_att, l0_bias, l0_res_W, l0_res_b,
           l1_Wl, l1_Wr, l1_att, l1_res_W, l1_res_b,
           dec_W, dec_b, cls_W, cls_b):
    f32 = jnp.float32
    src = edge_index[0].astype(jnp.int32)
    dst = edge_index[1].astype(jnp.int32)
    tf = t.astype(f32).reshape(1, 1)
    zeros = jnp.zeros((ROWS_PER_TILE, C), f32)
    att0f = l0_att.reshape(1, HID)
    att1f = l1_att.reshape(1, HID)

    r2 = lambda v: v.reshape(1, -1)

    cl0, cr0, cres0 = pl.pallas_call(
        _prep_body,
        out_shape=[jax.ShapeDtypeStruct((1, HID), f32)] * 3,
    )(tf, time_W0, r2(time_b0), time_W1, r2(time_b1),
      l0_Wl[NF:], r2(l0_bl), l0_Wr[NF:], r2(l0_br), l0_res_W[NF:], r2(l0_res_b))

    xlst0, xrst0, res0, snum0, sden0 = pl.pallas_call(
        _node0_body,
        grid=(NB,),
        in_specs=[_rows16] + [_full((NF, HID))] * 3 + [_full((1, HID))] * 4,
        out_specs=[_st16, _st16, _rows32, _st16, _st16],
        out_shape=[
            jax.ShapeDtypeStruct((2, N, C), f32),
            jax.ShapeDtypeStruct((2, N, C), f32),
            jax.ShapeDtypeStruct((N, HID), f32),
            jax.ShapeDtypeStruct((2, N, C), f32),
            jax.ShapeDtypeStruct((2, N, C), f32),
        ],
    )(x, l0_Wl[:NF], l0_Wr[:NF], l0_res_W[:NF], cl0, cr0, cres0, att0f)

    ex0, den0 = _sc_phase_a(xlst0.reshape(2 * N, C), xrst0.reshape(2 * N, C),
                            src, dst, l0_att.reshape(HID), zeros)
    num0 = _sc_phase_b(xlst0.reshape(2 * N, C), src, dst, ex0, zeros)

    xlst1, xrst1, res1, snum1, sden1 = pl.pallas_call(
        _node1_body,
        grid=(NB,),
        in_specs=[_st16, _st16, _st16, _st16, _rows32, _full((1, HID))]
        + [_full((HID, HID))] * 3 + [_full((1, HID))] * 2,
        out_specs=[_st16, _st16, _rows32, _st16, _st16],
        out_shape=[
            jax.ShapeDtypeStruct((2, N, C), f32),
            jax.ShapeDtypeStruct((2, N, C), f32),
            jax.ShapeDtypeStruct((N, HID), f32),
            jax.ShapeDtypeStruct((2, N, C), f32),
            jax.ShapeDtypeStruct((2, N, C), f32),
        ],
    )(num0.reshape(2, N, C), den0.reshape(2, N, C), snum0, sden0, res0,
      r2(l0_bias), l1_Wl, l1_Wr, l1_res_W, r2(l1_res_b), att1f)

    ex1, den1 = _sc_phase_a(xlst1.reshape(2 * N, C), xrst1.reshape(2 * N, C),
                            src, dst, l1_att.reshape(HID), zeros)
    num1 = _sc_phase_b(xlst1.reshape(2 * N, C), src, dst, ex1, zeros)

    logits = pl.pallas_call(
        _decode_body,
        grid=(NB,),
        in_specs=[_st16, _st16, _st16, _st16, _rows32,
                  _full((HID, C)), _full((1, C)), _full((C, 2)), _full((1, 2))],
        out_specs=pl.BlockSpec((R, 2), lambda i: (i, 0)),
        out_shape=jax.ShapeDtypeStruct((N, 2), f32),
    )(num1.reshape(2, N, C), den1.reshape(2, N, C), snum1, sden1, res1,
      dec_W, r2(dec_b), cls_W, r2(cls_b))

    return logits.reshape(1, N, 2)


# speedup vs baseline: 56.7173x; 1.9096x over previous
import dataclasses
import functools

import jax
import jax.numpy as jnp
from jax import lax
from jax.experimental import pallas as pl
from jax.experimental.pallas import tpu as pltpu
from jax.experimental.pallas import tpu_sc as plsc

N = 100000
E = 1600000
NF = 16
TDIM = 16
H = 2
C = 16
HID = H * C

NSUB = 16
EPT = E // NSUB
CH = 128
NCHUNK = EPT // CH
TAIL_E = EPT - NCHUNK * CH
ROWS_PER_TILE = 6248
TAIL_ROWS = N - NSUB * ROWS_PER_TILE
TAIL_OFF = NSUB * ROWS_PER_TILE


def _mesh():
    return plsc.VectorSubcoreMesh(core_axis_name="c", subcore_axis_name="s")


def _sc_params():
    cp = pltpu.CompilerParams()
    if "needs_layout_passes" in pltpu.CompilerParams.__dataclass_fields__:
        cp = dataclasses.replace(cp, needs_layout_passes=False)
    if "use_tc_tiling_on_sc" in pltpu.CompilerParams.__dataclass_fields__:
        cp = dataclasses.replace(cp, use_tc_tiling_on_sc=False)
    return cp


def _sc_phase_a(xlst, xrst, src, dst, attf, z2d):
    @functools.partial(
        pl.kernel,
        mesh=_mesh(),
        compiler_params=_sc_params(),
        out_type=[
            jax.ShapeDtypeStruct((2 * E,), jnp.float32),
            jax.ShapeDtypeStruct((2 * N, C), jnp.float32),
        ],
        scratch_types=[
            pltpu.VMEM((2, CH), jnp.int32),
            pltpu.VMEM((2, CH), jnp.int32),
            pltpu.VMEM((2, CH, C), jnp.float32),
            pltpu.VMEM((2, CH, C), jnp.float32),
            pltpu.VMEM((CH, C), jnp.float32),
            pltpu.VMEM((CH,), jnp.float32),
            pltpu.VMEM((C,), jnp.float32),
            pltpu.VMEM((TAIL_E,), jnp.int32),
            pltpu.VMEM((TAIL_E,), jnp.int32),
            pltpu.VMEM((TAIL_E, C), jnp.float32),
            pltpu.VMEM((TAIL_E, C), jnp.float32),
            pltpu.VMEM((TAIL_E, C), jnp.float32),
            pltpu.VMEM_SHARED((N, C), jnp.float32),
            pltpu.SemaphoreType.DMA,
            pltpu.SemaphoreType.DMA,
            pltpu.SemaphoreType.DMA,
            pltpu.SemaphoreType.DMA,
            pltpu.SemaphoreType.DMA,
            pltpu.SemaphoreType.DMA,
            pltpu.SemaphoreType.DMA,
            pltpu.SemaphoreType.DMA,
        ],
    )
    def kern(xl_hbm, xr_hbm, src_hbm, dst_hbm, att_hbm, z2_hbm,
             ex_hbm, den_hbm,
             srcb, dstb, xlb, xrb, denb, exb, attv,
             tailsrc, taildst, tailxl, tailxr, tailden, spden,
             s_s0, s_d0, s_l0, s_r0, s_s1, s_d1, s_l1, s_r1):
        cid = lax.axis_index("c")
        sid = lax.axis_index("s")
        ebase0 = sid * EPT
        xlv = xl_hbm.at[cid]
        xrv = xr_hbm.at[cid]

        pltpu.sync_copy(z2_hbm,
                        spden.at[pl.ds(sid * ROWS_PER_TILE, ROWS_PER_TILE)])

        @pl.when(sid == NSUB - 1)
        def _():
            pltpu.sync_copy(z2_hbm.at[pl.ds(0, TAIL_ROWS)],
                            spden.at[pl.ds(TAIL_OFF, TAIL_ROWS)])

        pltpu.sync_copy(att_hbm.at[pl.ds(cid * C, C)], attv)
        plsc.subcore_barrier()

        iota16 = lax.iota(jnp.int32, 16)
        att_sp = [plsc.load_gather(attv, [jnp.full((16,), c, jnp.int32)])
                  for c in range(C)]
        sems = ((s_s0, s_d0, s_l0, s_r0), (s_s1, s_d1, s_l1, s_r1))

        def idx_issue(i, b):
            eb = ebase0 + i * CH
            pltpu.async_copy(src_hbm.at[pl.ds(eb, CH)], srcb.at[b], sems[b][0])
            pltpu.async_copy(dst_hbm.at[pl.ds(eb, CH)], dstb.at[b], sems[b][1])

        def idx_wait(i, b):
            eb = ebase0 + i * CH
            pltpu.make_async_copy(src_hbm.at[pl.ds(eb, CH)], srcb.at[b],
                                  sems[b][0]).wait()
            pltpu.make_async_copy(dst_hbm.at[pl.ds(eb, CH)], dstb.at[b],
                                  sems[b][1]).wait()

        def gat_issue(b):
            pltpu.async_copy(xlv.at[srcb.at[b]], xlb.at[b], sems[b][2])
            pltpu.async_copy(xrv.at[dstb.at[b]], xrb.at[b], sems[b][3])

        def gat_wait(b):
            pltpu.make_async_copy(xlv.at[srcb.at[b]], xlb.at[b],
                                  sems[b][2]).wait()
            pltpu.make_async_copy(xrv.at[dstb.at[b]], xrb.at[b],
                                  sems[b][3]).wait()

        def groups(xl_b, xr_b, den_b, ngroups):
            for g in range(ngroups):
                edge16 = iota16 + g * 16
                acc = jnp.zeros((16,), jnp.float32)
                for c in range(C):
                    csp = jnp.full((16,), c, jnp.int32)
                    xlv16 = plsc.load_gather(xl_b, [edge16, csp])
                    xrv16 = plsc.load_gather(xr_b, [edge16, csp])
                    sv = xlv16 + xrv16
                    acc = acc + jnp.maximum(sv, 0.2 * sv) * att_sp[c]
                ex16 = jnp.exp(acc)
                exb[pl.ds(g * 16, 16)] = ex16
                for e in range(16):
                    esp = jnp.full((16,), g * 16 + e, jnp.int32)
                    den_b[g * 16 + e, :] = plsc.load_gather(exb, [esp])

        def compute(i, b):
            groups(xlb.at[b], xrb.at[b], denb, CH // 16)
            pltpu.sync_copy(denb, spden.at[dstb.at[b]], add=True)
            pltpu.sync_copy(exb,
                            ex_hbm.at[pl.ds(cid * E + ebase0 + i * CH, CH)])

        idx_issue(0, 0)
        idx_wait(0, 0)
        gat_issue(0)
        idx_issue(1, 1)

        @pl.loop(0, NCHUNK // 2)
        def _(p):
            for b in range(2):
                i = p * 2 + b
                nb = 1 - b
                idx_wait(i + 1, nb)
                gat_issue(nb)
                gat_wait(b)
                compute(i, b)

                @pl.when(i + 2 < NCHUNK)
                def _():
                    idx_issue(i + 2, b)

        gat_wait(0)
        compute(NCHUNK - 1, 0)

        tb = ebase0 + NCHUNK * CH
        pltpu.sync_copy(src_hbm.at[pl.ds(tb, TAIL_E)], tailsrc)
        pltpu.sync_copy(dst_hbm.at[pl.ds(tb, TAIL_E)], taildst)
        pltpu.sync_copy(xlv.at[tailsrc], tailxl)
        pltpu.sync_copy(xrv.at[taildst], tailxr)
        groups(tailxl, tailxr, tailden, TAIL_E // 16)
        pltpu.sync_copy(tailden, spden.at[taildst], add=True)
        pltpu.sync_copy(exb.at[pl.ds(0, TAIL_E)],
                        ex_hbm.at[pl.ds(cid * E + tb, TAIL_E)])

        plsc.subcore_barrier()
        r0 = sid * ROWS_PER_TILE
        pltpu.sync_copy(spden.at[pl.ds(r0, ROWS_PER_TILE)],
                        den_hbm.at[pl.ds(cid * N + r0, ROWS_PER_TILE)])

        @pl.when(sid == NSUB - 1)
        def _():
            pltpu.sync_copy(spden.at[pl.ds(TAIL_OFF, TAIL_ROWS)],
                            den_hbm.at[pl.ds(cid * N + TAIL_OFF, TAIL_ROWS)])

    return kern(xlst, xrst, src, dst, attf, z2d)


def _sc_phase_b(xlst, src, dst, ex, z2d):
    @functools.partial(
        pl.kernel,
        mesh=_mesh(),
        compiler_params=_sc_params(),
        out_type=jax.ShapeDtypeStruct((2 * N, C), jnp.float32),
        scratch_types=[
            pltpu.VMEM((2, CH), jnp.int32),
            pltpu.VMEM((2, CH), jnp.int32),
            pltpu.VMEM((2, CH, C), jnp.float32),
            pltpu.VMEM((2, CH), jnp.float32),
            pltpu.VMEM((CH, C), jnp.float32),
            pltpu.VMEM((TAIL_E,), jnp.int32),
            pltpu.VMEM((TAIL_E,), jnp.int32),
            pltpu.VMEM((TAIL_E,), jnp.float32),
            pltpu.VMEM((TAIL_E, C), jnp.float32),
            pltpu.VMEM((TAIL_E, C), jnp.float32),
            pltpu.VMEM_SHARED((N, C), jnp.float32),
            pltpu.SemaphoreType.DMA,
            pltpu.SemaphoreType.DMA,
            pltpu.SemaphoreType.DMA,
            pltpu.SemaphoreType.DMA,
            pltpu.SemaphoreType.DMA,
            pltpu.SemaphoreType.DMA,
            pltpu.SemaphoreType.DMA,
            pltpu.SemaphoreType.DMA,
        ],
    )
    def kern(xl_hbm, src_hbm, dst_hbm, exin_hbm, z2_hbm, num_hbm,
             srcb, dstb, xlb, exb, numb,
             tailsrc, taildst, tailex, tailxl, tailnum, spnum,
             s_s0, s_d0, s_e0, s_l0, s_s1, s_d1, s_e1, s_l1):
        cid = lax.axis_index("c")
        sid = lax.axis_index("s")
        ebase0 = sid * EPT
        xlv = xl_hbm.at[cid]

        pltpu.sync_copy(z2_hbm,
                        spnum.at[pl.ds(sid * ROWS_PER_TILE, ROWS_PER_TILE)])

        @pl.when(sid == NSUB - 1)
        def _():
            pltpu.sync_copy(z2_hbm.at[pl.ds(0, TAIL_ROWS)],
                            spnum.at[pl.ds(TAIL_OFF, TAIL_ROWS)])

        plsc.subcore_barrier()

        iota16 = lax.iota(jnp.int32, 16)
        sems = ((s_s0, s_d0, s_e0, s_l0), (s_s1, s_d1, s_e1, s_l1))

        def idx_issue(i, b):
            eb = ebase0 + i * CH
            pltpu.async_copy(src_hbm.at[pl.ds(eb, CH)], srcb.at[b], sems[b][0])
            pltpu.async_copy(dst_hbm.at[pl.ds(eb, CH)], dstb.at[b], sems[b][1])
            pltpu.async_copy(exin_hbm.at[pl.ds(cid * E + eb, CH)], exb.at[b],
                             sems[b][2])

        def idx_wait(i, b):
            eb = ebase0 + i * CH
            pltpu.make_async_copy(src_hbm.at[pl.ds(eb, CH)], srcb.at[b],
                                  sems[b][0]).wait()
            pltpu.make_async_copy(dst_hbm.at[pl.ds(eb, CH)], dstb.at[b],
                                  sems[b][1]).wait()
            pltpu.make_async_copy(exin_hbm.at[pl.ds(cid * E + eb, CH)],
                                  exb.at[b], sems[b][2]).wait()

        def gat_issue(b):
            pltpu.async_copy(xlv.at[srcb.at[b]], xlb.at[b], sems[b][3])

        def gat_wait(b):
            pltpu.make_async_copy(xlv.at[srcb.at[b]], xlb.at[b],
                                  sems[b][3]).wait()

        def groups(xl_b, ex_b, num_b, ngroups):
            for g in range(ngroups):
                for e in range(16):
                    esp = jnp.full((16,), g * 16 + e, jnp.int32)
                    vsp = plsc.load_gather(ex_b, [esp])
                    num_b[g * 16 + e, :] = xl_b[g * 16 + e, :] * vsp

        def compute(b):
            groups(xlb.at[b], exb.at[b], numb, CH // 16)
            pltpu.sync_copy(numb, spnum.at[dstb.at[b]], add=True)

        idx_issue(0, 0)
        idx_wait(0, 0)
        gat_issue(0)
        idx_issue(1, 1)

        @pl.loop(0, NCHUNK // 2)
        def _(p):
            for b in range(2):
                i = p * 2 + b
                nb = 1 - b
                idx_wait(i + 1, nb)
                gat_issue(nb)
                gat_wait(b)
                compute(b)

                @pl.when(i + 2 < NCHUNK)
                def _():
                    idx_issue(i + 2, b)

        gat_wait(0)
        compute(0)

        tb = ebase0 + NCHUNK * CH
        pltpu.sync_copy(src_hbm.at[pl.ds(tb, TAIL_E)], tailsrc)
        pltpu.sync_copy(dst_hbm.at[pl.ds(tb, TAIL_E)], taildst)
        pltpu.sync_copy(exin_hbm.at[pl.ds(cid * E + tb, TAIL_E)], tailex)
        pltpu.sync_copy(xlv.at[tailsrc], tailxl)
        groups(tailxl, tailex, tailnum, TAIL_E // 16)
        pltpu.sync_copy(tailnum, spnum.at[taildst], add=True)

        plsc.subcore_barrier()
        r0 = sid * ROWS_PER_TILE
        pltpu.sync_copy(spnum.at[pl.ds(r0, ROWS_PER_TILE)],
                        num_hbm.at[pl.ds(cid * N + r0, ROWS_PER_TILE)])

        @pl.when(sid == NSUB - 1)
        def _():
            pltpu.sync_copy(spnum.at[pl.ds(TAIL_OFF, TAIL_ROWS)],
                            num_hbm.at[pl.ds(cid * N + TAIL_OFF, TAIL_ROWS)])

    return kern(xlst, src, dst, ex, z2d)


R = 2048
NB = (N + R - 1) // R

_full = lambda shape: pl.BlockSpec(shape, lambda i: tuple(0 for _ in shape))
_rows32 = pl.BlockSpec((R, HID), lambda i: (i, 0))
_rows16 = pl.BlockSpec((R, NF), lambda i: (i, 0))
_st16 = pl.BlockSpec((2, R, C), lambda i: (0, i, 0))


def _prep_body(tf, w0, b0, w1, b1, wlt, bl, wrt, br, wrest, bres,
               cl_ref, cr_ref, cres_ref):
    ts = tf[0, 0]
    i8 = lax.broadcasted_iota(jnp.int32, (1, TDIM // 2), 1).astype(jnp.float32)
    freq = jnp.exp(i8 * (-jnp.log(10000.0) / (TDIM // 2 - 1)))
    emb = ts * freq
    emb = jnp.concatenate([jnp.sin(emb), jnp.cos(emb)], axis=1)
    temb = jax.nn.silu(jnp.dot(emb, w0[...], preferred_element_type=jnp.float32)
                       + b0[...])
    temb = jax.nn.silu(jnp.dot(temb, w1[...], preferred_element_type=jnp.float32)
                       + b1[...])
    cl_ref[...] = bl[...] + jnp.dot(temb, wlt[...], preferred_element_type=jnp.float32)
    cr_ref[...] = br[...] + jnp.dot(temb, wrt[...], preferred_element_type=jnp.float32)
    cres_ref[...] = bres[...] + jnp.dot(temb, wrest[...],
                                        preferred_element_type=jnp.float32)


def _self_terms(xl, xr, attf):
    s = xl + xr
    lv = jnp.maximum(s, 0.2 * s) * attf
    e0 = jnp.exp(jnp.sum(lv[:, :C], axis=1, keepdims=True))
    e1 = jnp.exp(jnp.sum(lv[:, C:], axis=1, keepdims=True))
    return e0, e1


def _node0_body(x, wlx, wrx, wresx, cl, cr, cres, attf,
                xlst, xrst, res0, snum, sden):
    xv = x[...]
    xl = jnp.dot(xv, wlx[...], preferred_element_type=jnp.float32) + cl[...]
    xr = jnp.dot(xv, wrx[...], preferred_element_type=jnp.float32) + cr[...]
    res0[...] = jax.nn.relu(
        jnp.dot(xv, wresx[...], preferred_element_type=jnp.float32) + cres[...])
    e0, e1 = _self_terms(xl, xr, attf[...])
    xlst[0] = xl[:, :C]
    xlst[1] = xl[:, C:]
    xrst[0] = xr[:, :C]
    xrst[1] = xr[:, C:]
    snum[0] = xl[:, :C] * e0
    snum[1] = xl[:, C:] * e1
    sden[0] = jnp.broadcast_to(e0, (R, C))
    sden[1] = jnp.broadcast_to(e1, (R, C))


def _node1_body(num, den, snum, sden, res0, bias0, wl, wr, wres, bres, attf,
                xlst, xrst, res1, snum1, sden1):
    o0 = (num[0] + snum[0]) / (den[0] + sden[0])
    o1 = (num[1] + snum[1]) / (den[1] + sden[1])
    hc = jax.nn.relu(jnp.concatenate([o0, o1], axis=1) + bias0[...])
    h = hc + res0[...]
    xl = jnp.dot(h, wl[...], preferred_element_type=jnp.float32)
    xr = jnp.dot(h, wr[...], preferred_element_type=jnp.float32)
    res1[...] = jax.nn.relu(
        jnp.dot(h, wres[...], preferred_element_type=jnp.float32) + bres[...])
    e0, e1 = _self_terms(xl, xr, attf[...])
    xlst[0] = xl[:, :C]
    xlst[1] = xl[:, C:]
    xrst[0] = xr[:, :C]
    xrst[1] = xr[:, C:]
    snum1[0] = xl[:, :C] * e0
    snum1[1] = xl[:, C:] * e1
    sden1[0] = jnp.broadcast_to(e0, (R, C))
    sden1[1] = jnp.broadcast_to(e1, (R, C))


def _decode_body(num, den, snum, sden, res1, decw, decb, clsw, clsb, out):
    o0 = (num[0] + snum[0]) / (den[0] + sden[0])
    o1 = (num[1] + snum[1]) / (den[1] + sden[1])
    h = jax.nn.relu(jnp.concatenate([o0, o1], axis=1)) + res1[...]
    d = jnp.dot(h, decw[...], preferred_element_type=jnp.float32) + decb[...]
    out[...] = jnp.dot(d, clsw[...], preferred_element_type=jnp.float32) + clsb[...]


def kernel(x, edge_index, t, time_W0, time_b0, time_W1, time_b1,
           l0_Wl, l0_bl, l0_Wr, l0_br, l0_att, l0_bias, l0_res_W, l0_res_b,
           l1_Wl, l1_Wr, l1_att, l1_res_W, l1_res_b,
           dec_W, dec_b, cls_W, cls_b):
    f32 = jnp.float32
    src = edge_index[0].astype(jnp.int32)
    dst = edge_index[1].astype(jnp.int32)
    tf = t.astype(f32).reshape(1, 1)
    z2d = jnp.zeros((ROWS_PER_TILE, C), f32)
    att0f = l0_att.reshape(1, HID)
    att1f = l1_att.reshape(1, HID)

    r2 = lambda v: v.reshape(1, -1)

    cl0, cr0, cres0 = pl.pallas_call(
        _prep_body,
        out_shape=[jax.ShapeDtypeStruct((1, HID), f32)] * 3,
    )(tf, time_W0, r2(time_b0), time_W1, r2(time_b1),
      l0_Wl[NF:], r2(l0_bl), l0_Wr[NF:], r2(l0_br), l0_res_W[NF:], r2(l0_res_b))

    xlst0, xrst0, res0, snum0, sden0 = pl.pallas_call(
        _node0_body,
        grid=(NB,),
        in_specs=[_rows16] + [_full((NF, HID))] * 3 + [_full((1, HID))] * 4,
        out_specs=[_st16, _st16, _rows32, _st16, _st16],
        out_shape=[
            jax.ShapeDtypeStruct((2, N, C), f32),
            jax.ShapeDtypeStruct((2, N, C), f32),
            jax.ShapeDtypeStruct((N, HID), f32),
            jax.ShapeDtypeStruct((2, N, C), f32),
            jax.ShapeDtypeStruct((2, N, C), f32),
        ],
    )(x, l0_Wl[:NF], l0_Wr[:NF], l0_res_W[:NF], cl0, cr0, cres0, att0f)

    ex0, den0 = _sc_phase_a(xlst0, xrst0, src, dst, l0_att.reshape(HID), z2d)
    num0 = _sc_phase_b(xlst0, src, dst, ex0, z2d)

    xlst1, xrst1, res1, snum1, sden1 = pl.pallas_call(
        _node1_body,
        grid=(NB,),
        in_specs=[_st16, _st16, _st16, _st16, _rows32, _full((1, HID))]
        + [_full((HID, HID))] * 3 + [_full((1, HID))] * 2,
        out_specs=[_st16, _st16, _rows32, _st16, _st16],
        out_shape=[
            jax.ShapeDtypeStruct((2, N, C), f32),
            jax.ShapeDtypeStruct((2, N, C), f32),
            jax.ShapeDtypeStruct((N, HID), f32),
            jax.ShapeDtypeStruct((2, N, C), f32),
            jax.ShapeDtypeStruct((2, N, C), f32),
        ],
    )(num0.reshape(2, N, C), den0.reshape(2, N, C), snum0, sden0, res0,
      r2(l0_bias), l1_Wl, l1_Wr, l1_res_W, r2(l1_res_b), att1f)

    ex1, den1 = _sc_phase_a(xlst1, xrst1, src, dst, l1_att.reshape(HID), z2d)
    num1 = _sc_phase_b(xlst1, src, dst, ex1, z2d)

    logits = pl.pallas_call(
        _decode_body,
        grid=(NB,),
        in_specs=[_st16, _st16, _st16, _st16, _rows32,
                  _full((HID, C)), _full((1, C)), _full((C, 2)), _full((1, 2))],
        out_specs=pl.BlockSpec((R, 2), lambda i: (i, 0)),
        out_shape=jax.ShapeDtypeStruct((N, 2), f32),
    )(num1.reshape(2, N, C), den1.reshape(2, N, C), snum1, sden1, res1,
      dec_W, r2(dec_b), cls_W, r2(cls_b))

    return logits.reshape(1, N, 2)
